# baseline (speedup 1.0000x reference)
import jax
import jax.numpy as jnp
from jax.experimental import pallas as pl

K_NEIGHBORS = 75
N_CLASSES = 10

N_TRAIN = 100000
N_PAD = 102400
QB1 = 256
NBLK = 4096
CMW = NBLK // 128
N_CHUNK = N_PAD // CMW
QB2 = 16
ROWBLK = 800


def _norm_body(ta_ref, tan_ref, sum_ref):
    i = pl.program_id(0)
    t = ta_ref[...]
    norm = jnp.sqrt(jnp.sum(t * t, axis=1, keepdims=True))
    tn = t / norm
    tan_ref[...] = tn
    part = jnp.sum(tn.reshape(ROWBLK // 8, 8, 128), axis=0)

    @pl.when(i == 0)
    def _():
        sum_ref[...] = part

    @pl.when(i > 0)
    def _():
        sum_ref[...] = sum_ref[...] + part


def _dist_body(q_ref, tan_ref, sum_ref, key_ref):
    c = jnp.sum(sum_ref[...], axis=0, keepdims=True) / float(N_TRAIN)
    q = q_ref[...]
    qn = q / jnp.sqrt(jnp.sum(q * q, axis=1, keepdims=True))
    qc = qn - c
    qq = jnp.sum(qc * qc, axis=1, keepdims=True)
    tac = tan_ref[...] - c
    tt = jnp.sum(tac * tac, axis=1, keepdims=True)
    s = jax.lax.dot_general(qc, tac, (((1,), (1,)), ((), ())),
                            preferred_element_type=jnp.float32)
    d2 = (qq - 2.0 * s) + tt.reshape(1, NBLK)
    j = pl.program_id(1)
    col = j * NBLK + jax.lax.broadcasted_iota(jnp.int32, d2.shape, 1)
    key_ref[...] = jnp.where(col < N_TRAIN, d2, 1e9)


def _search(count_lt, target, lo0, hi0, cl0, ch0, cap):
    tgt = float(target)

    def cond(carry):
        return (carry[0] < cap) & ~jnp.all(carry[6] > 0.5)

    def body(carry):
        i, lo, hi, cl, ch, tb, done = carry
        w = hi - lo
        frac = jnp.clip((tgt - cl) / (ch - cl), 0.02, 0.98)
        xs = lo + w * frac
        thrs = [lo + w * 0.2, lo + w * 0.4, lo + w * 0.6, lo + w * 0.8,
                xs - w * (1.0 / 64.0), xs - w * (1.0 / 512.0),
                xs + w * (1.0 / 512.0), xs + w * (1.0 / 64.0)]
        cnts = [count_lt(t).astype(jnp.float32) for t in thrs]
        thr_m = jnp.concatenate(thrs, axis=1)
        cnt_m = jnp.concatenate(cnts, axis=1)
        is_hit = cnt_m == tgt
        any_hit = jnp.any(is_hit, axis=1, keepdims=True)
        t_hit = jnp.min(jnp.where(is_hit, thr_m, 1e9), axis=1,
                        keepdims=True)
        new_tb = jnp.where((done < 0.5) & any_hit, t_hit, tb)
        new_done = jnp.maximum(done, any_hit.astype(jnp.float32))
        lt = cnt_m < tgt
        gt = cnt_m > tgt
        new_lo = jnp.maximum(lo, jnp.max(
            jnp.where(lt, thr_m, -1e9), axis=1, keepdims=True))
        new_cl = jnp.maximum(cl, jnp.max(
            jnp.where(lt, cnt_m, -1.0), axis=1, keepdims=True))
        new_hi = jnp.minimum(hi, jnp.min(
            jnp.where(gt, thr_m, 1e9), axis=1, keepdims=True))
        new_ch = jnp.minimum(ch, jnp.min(
            jnp.where(gt, cnt_m, 1e9), axis=1, keepdims=True))
        return (i + 1, new_lo, new_hi, new_cl, new_ch, new_tb, new_done)

    tb0 = jnp.zeros_like(lo0)
    done0 = jnp.zeros_like(lo0)
    carry = (jnp.int32(0), lo0, hi0, cl0, ch0, tb0, done0)
    _, lo, hi, cl, ch, tb, done = jax.lax.while_loop(cond, body, carry)
    return jnp.where(done > 0.5, tb, hi), done


def _select_body(key_ref, lab_ref, cali_ref, out_ref):
    keys = key_ref[...]
    lab = lab_ref[...]
    cmin = jnp.min(keys.reshape(QB2, CMW, N_CHUNK), axis=1)
    ones_col = jnp.ones((N_PAD, 1), jnp.float32)

    def count_keys(thr):
        mask = jnp.where(keys < thr, 1.0, 0.0)
        return jax.lax.dot_general(mask, ones_col, (((1,), (0,)), ((), ())),
                                   preferred_element_type=jnp.float32)

    def count_cmin(thr):
        return jnp.sum(cmin < thr, axis=1, keepdims=True)

    lo_cm = jnp.min(cmin, axis=1, keepdims=True)
    hi_cm = jnp.full((QB2, 1), 17.0, jnp.float32)
    t_cm, _ = _search(count_cmin, K_NEIGHBORS + 1,
                      lo_cm, hi_cm,
                      jnp.zeros((QB2, 1), jnp.float32),
                      jnp.full((QB2, 1), float(N_TRAIN // CMW),
                               jnp.float32), 8)

    t_final, _ = _search(count_keys, K_NEIGHBORS,
                         lo_cm, t_cm,
                         jnp.zeros((QB2, 1), jnp.float32),
                         jnp.full((QB2, 1), float(K_NEIGHBORS + 1),
                                  jnp.float32), 12)

    mask = keys < t_final
    g_id = lab // 3
    f_id = lab - 3 * g_id
    powf = jnp.where(f_id == 0, 1.0,
                     jnp.where(f_id == 1, 256.0, 65536.0))
    in_cols = []
    for g in range(4):
        pw = jnp.where(g_id == g, powf, 0.0)
        acc = jnp.sum(jnp.where(mask, pw, 0.0), axis=1, keepdims=True)
        c2 = jnp.floor(acc * (1.0 / 65536.0))
        r = acc - c2 * 65536.0
        c1 = jnp.floor(r * (1.0 / 256.0))
        c0 = r - c1 * 256.0
        if g < 3:
            in_cols += [c0, c1, c2]
        else:
            in_cols += [c0]
    in_class = jnp.concatenate(in_cols, axis=1)
    total = jnp.sum(in_class, axis=1, keepdims=True)
    not_in_class = (total - in_class).astype(jnp.int32)

    cali = cali_ref[...]
    nb_cali = float(cali.shape[1])
    pos_c = []
    for c in range(N_CLASSES):
        v = not_in_class[:, c:c + 1]
        pos_c.append(jnp.sum((cali < v).astype(jnp.float32), axis=1,
                             keepdims=True))
    pos = jnp.concatenate(pos_c, axis=1)
    p_value = (nb_cali - pos) / nb_cali

    best = jnp.max(p_value, axis=1, keepdims=True)
    cls = jax.lax.broadcasted_iota(jnp.int32, p_value.shape, 1)
    pred = jnp.min(jnp.where(p_value == best, cls, N_CLASSES + 1), axis=1,
                   keepdims=True)
    out_ref[...] = jnp.where(cls == pred, best, 0.0)


def kernel(queries, train_activations, train_labels, cali_nonconformity):
    nq = queries.shape[0]
    tan_pad, colsum = pl.pallas_call(
        _norm_body,
        grid=(N_TRAIN // ROWBLK,),
        in_specs=[pl.BlockSpec((ROWBLK, 128), lambda i: (i, 0))],
        out_specs=[
            pl.BlockSpec((ROWBLK, 128), lambda i: (i, 0)),
            pl.BlockSpec((8, 128), lambda i: (0, 0)),
        ],
        out_shape=[
            jax.ShapeDtypeStruct((N_PAD, 128), jnp.float32),
            jax.ShapeDtypeStruct((8, 128), jnp.float32),
        ],
    )(train_activations)

    keys = pl.pallas_call(
        _dist_body,
        grid=(nq // QB1, N_PAD // NBLK),
        in_specs=[
            pl.BlockSpec((QB1, 128), lambda i, j: (i, 0)),
            pl.BlockSpec((NBLK, 128), lambda i, j: (j, 0)),
            pl.BlockSpec((8, 128), lambda i, j: (0, 0)),
        ],
        out_specs=pl.BlockSpec((QB1, NBLK), lambda i, j: (i, j)),
        out_shape=jax.ShapeDtypeStruct((nq, N_PAD), jnp.float32),
    )(queries, tan_pad, colsum)

    lab_pad = jnp.pad(train_labels, (0, N_PAD - N_TRAIN),
                      constant_values=N_CLASSES).reshape(1, N_PAD)
    cali2d = cali_nonconformity.reshape(1, -1)

    creds = pl.pallas_call(
        _select_body,
        grid=(nq // QB2,),
        in_specs=[
            pl.BlockSpec((QB2, N_PAD), lambda i: (i, 0)),
            pl.BlockSpec((1, N_PAD), lambda i: (0, 0)),
            pl.BlockSpec((1, cali2d.shape[1]), lambda i: (0, 0)),
        ],
        out_specs=pl.BlockSpec((QB2, N_CLASSES), lambda i: (i, 0)),
        out_shape=jax.ShapeDtypeStruct((nq, N_CLASSES), jnp.float32),
    )(keys, lab_pad, cali2d)
    return creds


# speedup vs baseline: 3.5888x; 3.5888x over previous
import jax
import jax.numpy as jnp
from jax.experimental import pallas as pl

K_NEIGHBORS = 75
N_CLASSES = 10

N_TRAIN = 100000
N_PAD = 102400
QB1 = 256
NBLK = 4096
CMW = NBLK // 128
N_CHUNK = N_PAD // CMW
QB2 = 16
ROWBLK = 800


def _norm_body(ta_ref, tan_ref, sum_ref):
    i = pl.program_id(0)
    t = ta_ref[...]
    norm = jnp.sqrt(jnp.sum(t * t, axis=1, keepdims=True))
    tn = t / norm
    tan_ref[...] = tn
    part = jnp.sum(tn.reshape(ROWBLK // 8, 8, 128), axis=0)

    @pl.when(i == 0)
    def _():
        sum_ref[...] = part

    @pl.when(i > 0)
    def _():
        sum_ref[...] = sum_ref[...] + part


def _dist_body(q_ref, tan_ref, sum_ref, key_ref, cmin_ref):
    c = jnp.sum(sum_ref[...], axis=0, keepdims=True) / float(N_TRAIN)
    q = q_ref[...]
    qn = q / jnp.sqrt(jnp.sum(q * q, axis=1, keepdims=True))
    qc = qn - c
    qq = jnp.sum(qc * qc, axis=1, keepdims=True)
    tac = tan_ref[...] - c
    tt = jnp.sum(tac * tac, axis=1, keepdims=True)
    s = jax.lax.dot_general(qc, tac, (((1,), (1,)), ((), ())),
                            preferred_element_type=jnp.float32)
    d2 = (qq - 2.0 * s) + tt.reshape(1, NBLK)
    j = pl.program_id(1)
    col = j * NBLK + jax.lax.broadcasted_iota(jnp.int32, d2.shape, 1)
    key = jnp.where(col < N_TRAIN, d2, 1e9)
    key_ref[...] = key
    cmin_ref[...] = jnp.min(key.reshape(QB1, CMW, 128), axis=1)


def _search(count_lt, target, lo0, hi0, cl0, ch0, cap):
    tgt = float(target)

    def cond(carry):
        return (carry[0] < cap) & ~jnp.all(carry[6] > 0.5)

    def body(carry):
        i, lo, hi, cl, ch, tb, done = carry
        w = hi - lo
        frac = jnp.clip((tgt - cl) / (ch - cl), 0.02, 0.98)
        xs = lo + w * frac
        thrs = [lo + w * 0.2, lo + w * 0.4, lo + w * 0.6, lo + w * 0.8,
                xs - w * (1.0 / 64.0), xs - w * (1.0 / 512.0),
                xs + w * (1.0 / 512.0), xs + w * (1.0 / 64.0)]
        cnts = [count_lt(t).astype(jnp.float32) for t in thrs]
        thr_m = jnp.concatenate(thrs, axis=1)
        cnt_m = jnp.concatenate(cnts, axis=1)
        is_hit = cnt_m == tgt
        any_hit = jnp.any(is_hit, axis=1, keepdims=True)
        t_hit = jnp.min(jnp.where(is_hit, thr_m, 1e9), axis=1,
                        keepdims=True)
        new_tb = jnp.where((done < 0.5) & any_hit, t_hit, tb)
        new_done = jnp.maximum(done, any_hit.astype(jnp.float32))
        lt = cnt_m < tgt
        gt = cnt_m > tgt
        new_lo = jnp.maximum(lo, jnp.max(
            jnp.where(lt, thr_m, -1e9), axis=1, keepdims=True))
        new_cl = jnp.maximum(cl, jnp.max(
            jnp.where(lt, cnt_m, -1.0), axis=1, keepdims=True))
        new_hi = jnp.minimum(hi, jnp.min(
            jnp.where(gt, thr_m, 1e9), axis=1, keepdims=True))
        new_ch = jnp.minimum(ch, jnp.min(
            jnp.where(gt, cnt_m, 1e9), axis=1, keepdims=True))
        return (i + 1, new_lo, new_hi, new_cl, new_ch, new_tb, new_done)

    tb0 = jnp.zeros_like(lo0)
    done0 = jnp.zeros_like(lo0)
    carry = (jnp.int32(0), lo0, hi0, cl0, ch0, tb0, done0)
    _, lo, hi, cl, ch, tb, done = jax.lax.while_loop(cond, body, carry)
    return jnp.where(done > 0.5, tb, hi), done


def _select_body(key_ref, cmin_ref, lab_ref, cali_ref, out_ref):
    keys = key_ref[...]
    cmin = cmin_ref[...]
    lab = lab_ref[...]

    def count_keys(thr):
        return jnp.sum(keys < thr, axis=1, keepdims=True)

    def count_cmin(thr):
        return jnp.sum(cmin < thr, axis=1, keepdims=True)

    lo_cm = jnp.min(cmin, axis=1, keepdims=True)
    hi_cm = jnp.full((QB2, 1), 17.0, jnp.float32)
    t_cm, _ = _search(count_cmin, K_NEIGHBORS + 1,
                      lo_cm, hi_cm,
                      jnp.zeros((QB2, 1), jnp.float32),
                      jnp.full((QB2, 1), float(N_TRAIN // CMW),
                               jnp.float32), 8)

    t_final, _ = _search(count_keys, K_NEIGHBORS,
                         lo_cm, t_cm,
                         jnp.zeros((QB2, 1), jnp.float32),
                         jnp.full((QB2, 1), float(K_NEIGHBORS + 1),
                                  jnp.float32), 12)

    mask = keys < t_final
    g_id = lab // 3
    f_id = lab - 3 * g_id
    powf = jnp.where(f_id == 0, 1.0,
                     jnp.where(f_id == 1, 256.0, 65536.0))
    in_cols = []
    for g in range(4):
        pw = jnp.where(g_id == g, powf, 0.0)
        acc = jnp.sum(jnp.where(mask, pw, 0.0), axis=1, keepdims=True)
        c2 = jnp.floor(acc * (1.0 / 65536.0))
        r = acc - c2 * 65536.0
        c1 = jnp.floor(r * (1.0 / 256.0))
        c0 = r - c1 * 256.0
        if g < 3:
            in_cols += [c0, c1, c2]
        else:
            in_cols += [c0]
    in_class = jnp.concatenate(in_cols, axis=1)
    total = jnp.sum(in_class, axis=1, keepdims=True)
    not_in_class = (total - in_class).astype(jnp.int32)

    cali = cali_ref[...]
    nb_cali = float(cali.shape[1])
    pos_c = []
    for c in range(N_CLASSES):
        v = not_in_class[:, c:c + 1]
        pos_c.append(jnp.sum((cali < v).astype(jnp.float32), axis=1,
                             keepdims=True))
    pos = jnp.concatenate(pos_c, axis=1)
    p_value = (nb_cali - pos) / nb_cali

    best = jnp.max(p_value, axis=1, keepdims=True)
    cls = jax.lax.broadcasted_iota(jnp.int32, p_value.shape, 1)
    pred = jnp.min(jnp.where(p_value == best, cls, N_CLASSES + 1), axis=1,
                   keepdims=True)
    out_ref[...] = jnp.where(cls == pred, best, 0.0)


def kernel(queries, train_activations, train_labels, cali_nonconformity):
    nq = queries.shape[0]
    tan_pad, colsum = pl.pallas_call(
        _norm_body,
        grid=(N_TRAIN // ROWBLK,),
        in_specs=[pl.BlockSpec((ROWBLK, 128), lambda i: (i, 0))],
        out_specs=[
            pl.BlockSpec((ROWBLK, 128), lambda i: (i, 0)),
            pl.BlockSpec((8, 128), lambda i: (0, 0)),
        ],
        out_shape=[
            jax.ShapeDtypeStruct((N_PAD, 128), jnp.float32),
            jax.ShapeDtypeStruct((8, 128), jnp.float32),
        ],
    )(train_activations)

    keys, cmin = pl.pallas_call(
        _dist_body,
        grid=(nq // QB1, N_PAD // NBLK),
        in_specs=[
            pl.BlockSpec((QB1, 128), lambda i, j: (i, 0)),
            pl.BlockSpec((NBLK, 128), lambda i, j: (j, 0)),
            pl.BlockSpec((8, 128), lambda i, j: (0, 0)),
        ],
        out_specs=[
            pl.BlockSpec((QB1, NBLK), lambda i, j: (i, j)),
            pl.BlockSpec((QB1, 128), lambda i, j: (i, j)),
        ],
        out_shape=[
            jax.ShapeDtypeStruct((nq, N_PAD), jnp.float32),
            jax.ShapeDtypeStruct((nq, N_CHUNK), jnp.float32),
        ],
    )(queries, tan_pad, colsum)

    lab_pad = jnp.pad(train_labels, (0, N_PAD - N_TRAIN),
                      constant_values=N_CLASSES).reshape(1, N_PAD)
    cali2d = cali_nonconformity.reshape(1, -1)

    creds = pl.pallas_call(
        _select_body,
        grid=(nq // QB2,),
        in_specs=[
            pl.BlockSpec((QB2, N_PAD), lambda i: (i, 0)),
            pl.BlockSpec((QB2, N_CHUNK), lambda i: (i, 0)),
            pl.BlockSpec((1, N_PAD), lambda i: (0, 0)),
            pl.BlockSpec((1, cali2d.shape[1]), lambda i: (0, 0)),
        ],
        out_specs=pl.BlockSpec((QB2, N_CLASSES), lambda i: (i, 0)),
        out_shape=jax.ShapeDtypeStruct((nq, N_CLASSES), jnp.float32),
    )(keys, cmin, lab_pad, cali2d)
    return creds
